# baseline (speedup 1.0000x reference)
import functools

import jax
import jax.numpy as jnp
from jax import lax
from jax.experimental import pallas as pl
from jax.experimental.pallas import tpu as pltpu
from jax.experimental.pallas import tpu_sc as plsc

B = 16
S = 576
T = B * S
D = 64
V = 8192

BB = 2
TB = BB * S
TSB = 128
N_TB = T // TB
NR = T // 128

NC, NS = 2, 16
NW = NC * NS


def _argmin_body(x_ref, e_ref, idx_ref, epad_ref, e2_ref):
    @pl.when(pl.program_id(0) == 0)
    def _():
        ef = e_ref[...]
        e2 = jnp.sum(ef * ef, axis=1)
        e2_ref[...] = e2.reshape(V // 128, 128)
        epad_ref[...] = jnp.concatenate(
            [ef, jnp.zeros((V, 128 - D), jnp.float32)], axis=1)

    x = x_ref[...].reshape(TB, D)
    raw = jax.lax.dot_general(
        x, e_ref[...], (((1,), (1,)), ((), ())),
        preferred_element_type=jnp.float32)
    lane = jax.lax.broadcasted_iota(jnp.int32, (TSB, 128), 1).astype(jnp.float32)
    for s in range(TB // TSB):
        best_val = jnp.full((TSB, 128), jnp.inf, dtype=jnp.float32)
        best_gid = jnp.zeros((TSB, 128), dtype=jnp.float32)
        rs = raw[s * TSB:(s + 1) * TSB, :]
        for G in range(V // 128):
            dg = -2.0 * rs[:, G * 128:(G + 1) * 128] + e2_ref[G][None, :]
            lt = dg < best_val
            best_gid = jnp.where(lt, jnp.float32(G), best_gid)
            best_val = jnp.minimum(dg, best_val)
        m = jnp.min(best_val, axis=1)
        cand = jnp.where(best_val == m[:, None],
                         best_gid * 128.0 + lane, jnp.float32(1e9))
        gr = (TB // TSB) * pl.program_id(0) + s
        idx_ref[pl.ds(gr, 1), :] = (
            jnp.min(cand, axis=1).astype(jnp.int32)[None, :])


def _nearest_idx(x, E):
    return pl.pallas_call(
        _argmin_body,
        grid=(N_TB,),
        in_specs=[
            pl.BlockSpec((BB, S, D), lambda i: (i, 0, 0)),
            pl.BlockSpec((V, D), lambda i: (0, 0)),
        ],
        out_specs=[
            pl.BlockSpec((NR, 128), lambda i: (0, 0)),
            pl.BlockSpec((V, 128), lambda i: (0, 0)),
        ],
        out_shape=[
            jax.ShapeDtypeStruct((NR, 128), jnp.int32),
            jax.ShapeDtypeStruct((V, 128), jnp.float32),
        ],
        scratch_shapes=[pltpu.VMEM((V // 128, 128), jnp.float32)],
    )(x, E)


def _sc_gather(epad, idx):
    mesh = plsc.VectorSubcoreMesh(core_axis_name="c", subcore_axis_name="s")

    @functools.partial(
        pl.kernel, mesh=mesh,
        out_type=jax.ShapeDtypeStruct((T, 128), jnp.float32),
        scratch_types=[
            pltpu.VMEM((8, 128), jnp.int32),
            pltpu.VMEM((3, 128, 128), jnp.float32),
            pltpu.SemaphoreType.DMA,
        ],
    )
    def gather_k(table_hbm, idx_hbm, out_hbm, tile_v, rows_v, sem):
        w = lax.axis_index("s") * NC + lax.axis_index("c")
        pltpu.sync_copy(idx_hbm.at[pl.ds((w // 4) * 8, 8)], tile_v)
        r0 = (2 * w) % 8
        cp0 = pltpu.async_copy(table_hbm.at[tile_v.at[r0]], rows_v.at[0], sem)
        cp1 = pltpu.async_copy(table_hbm.at[tile_v.at[r0 + 1]], rows_v.at[1], sem)
        cp0.wait()
        pltpu.sync_copy(rows_v.at[0], out_hbm.at[pl.ds((2 * w) * 128, 128)])
        cp1.wait()
        pltpu.sync_copy(rows_v.at[1], out_hbm.at[pl.ds((2 * w + 1) * 128, 128)])

        @pl.when(w < 8)
        def _():
            pltpu.sync_copy(idx_hbm.at[pl.ds(64, 8)], tile_v)
            pltpu.async_copy(
                table_hbm.at[tile_v.at[w]], rows_v.at[2], sem).wait()
            pltpu.sync_copy(rows_v.at[2],
                            out_hbm.at[pl.ds((64 + w) * 128, 128)])

    return gather_k(epad, idx)


def kernel(x, E):
    idx, epad = _nearest_idx(x, E)
    values = _sc_gather(epad, idx)
    return values[:, :D].reshape(B, S, D)


# speedup vs baseline: 1.2104x; 1.2104x over previous
import functools

import jax
import jax.numpy as jnp
from jax import lax
from jax.experimental import pallas as pl
from jax.experimental.pallas import tpu as pltpu
from jax.experimental.pallas import tpu_sc as plsc

B = 16
S = 576
T = B * S
D = 64
V = 8192

BB = 2
TB = BB * S
TSB = 128
N_TB = T // TB
NR = T // 128

NC, NS = 2, 16
NW = NC * NS
BPW = T // NW
CH = 96


def _argmin_body(x_ref, e_ref, idx_ref, epad_ref, e2_ref):
    @pl.when(pl.program_id(0) == 0)
    def _():
        ef = e_ref[...]
        e2 = jnp.sum(ef * ef, axis=1)
        e2_ref[...] = e2.reshape(V // 128, 128)
        epad_ref[...] = jnp.concatenate(
            [ef, jnp.zeros((V, 128 - D), jnp.float32)], axis=1)

    x = x_ref[...].reshape(TB, D) * -2.0
    raw = jax.lax.dot_general(
        x, e_ref[...], (((1,), (1,)), ((), ())),
        preferred_element_type=jnp.float32)
    lane = jax.lax.broadcasted_iota(jnp.int32, (TSB, 128), 1).astype(jnp.float32)
    for s in range(TB // TSB):
        best_val = jnp.full((TSB, 128), jnp.inf, dtype=jnp.float32)
        best_gid = jnp.zeros((TSB, 128), dtype=jnp.float32)
        rs = raw[s * TSB:(s + 1) * TSB, :]
        for G in range(V // 128):
            dg = rs[:, G * 128:(G + 1) * 128] + e2_ref[G][None, :]
            lt = dg < best_val
            best_gid = jnp.where(lt, jnp.float32(G), best_gid)
            best_val = jnp.minimum(dg, best_val)
        m = jnp.min(best_val, axis=1)
        cand = jnp.where(best_val == m[:, None],
                         best_gid * 128.0 + lane, jnp.float32(1e9))
        gr = (TB // TSB) * pl.program_id(0) + s
        idx_ref[pl.ds(gr, 1), :] = (
            jnp.min(cand, axis=1).astype(jnp.int32)[None, :])


def _nearest_idx(x, E):
    return pl.pallas_call(
        _argmin_body,
        grid=(N_TB,),
        in_specs=[
            pl.BlockSpec((BB, S, D), lambda i: (i, 0, 0)),
            pl.BlockSpec((V, D), lambda i: (0, 0)),
        ],
        out_specs=[
            pl.BlockSpec((NR, 128), lambda i: (0, 0)),
            pl.BlockSpec((V, 128), lambda i: (0, 0)),
        ],
        out_shape=[
            jax.ShapeDtypeStruct((NR, 128), jnp.int32),
            jax.ShapeDtypeStruct((V, 128), jnp.float32),
        ],
        scratch_shapes=[pltpu.VMEM((V // 128, 128), jnp.float32)],
    )(x, E)


def _sc_gather(epad, idx):
    mesh = plsc.VectorSubcoreMesh(core_axis_name="c", subcore_axis_name="s")

    @functools.partial(
        pl.kernel, mesh=mesh,
        out_type=jax.ShapeDtypeStruct((T, 128), jnp.float32),
        scratch_types=[
            pltpu.VMEM((BPW,), jnp.int32),
            pltpu.VMEM((BPW, 128), jnp.float32),
            pltpu.SemaphoreType.DMA,
        ],
    )
    def gather_k(table_hbm, idx_hbm, out_hbm, idx_v, rows_v, sem):
        w = lax.axis_index("s") * NC + lax.axis_index("c")
        pltpu.sync_copy(idx_hbm.at[pl.ds(w * BPW, BPW)], idx_v)
        copies = [
            pltpu.async_copy(table_hbm.at[idx_v.at[pl.ds(j * CH, CH)]],
                             rows_v.at[pl.ds(j * CH, CH)], sem)
            for j in range(BPW // CH)
        ]
        for cp in copies:
            cp.wait()
        pltpu.sync_copy(rows_v, out_hbm.at[pl.ds(w * BPW, BPW)])

    return gather_k(epad, idx)


def kernel(x, E):
    idx, epad = _nearest_idx(x, E)
    values = _sc_gather(epad, idx.reshape(T))
    return values[:, :D].reshape(B, S, D)


# speedup vs baseline: 1.2105x; 1.0001x over previous
import functools

import jax
import jax.numpy as jnp
from jax import lax
from jax.experimental import pallas as pl
from jax.experimental.pallas import tpu as pltpu
from jax.experimental.pallas import tpu_sc as plsc

B = 16
S = 576
T = B * S
D = 64
V = 8192

BB = 2
TB = BB * S
TSB = 64
N_TB = T // TB
NR = T // 128

NC, NS = 2, 16
NW = NC * NS
BPW = T // NW
CH = 96


def _argmin_body(x_ref, e_ref, idx_ref, epad_ref, e2_ref):
    @pl.when(pl.program_id(0) == 0)
    def _():
        ef = e_ref[...]
        e2 = jnp.sum(ef * ef, axis=1)
        e2_ref[...] = e2.reshape(V // 128, 128)
        epad_ref[...] = jnp.concatenate(
            [ef, jnp.zeros((V, 128 - D), jnp.float32)], axis=1)

    x = x_ref[...].reshape(TB, D) * -2.0
    raw = jax.lax.dot_general(
        x, e_ref[...], (((1,), (1,)), ((), ())),
        preferred_element_type=jnp.float32)
    lane = jax.lax.broadcasted_iota(jnp.int32, (TSB, 128), 1).astype(jnp.float32)
    for s in range(TB // TSB):
        best_val = jnp.full((TSB, 128), jnp.inf, dtype=jnp.float32)
        best_gid = jnp.zeros((TSB, 128), dtype=jnp.float32)
        rs = raw[s * TSB:(s + 1) * TSB, :]
        for G in range(V // 128):
            dg = rs[:, G * 128:(G + 1) * 128] + e2_ref[G][None, :]
            lt = dg < best_val
            best_gid = jnp.where(lt, jnp.float32(G), best_gid)
            best_val = jnp.minimum(dg, best_val)
        m = jnp.min(best_val, axis=1)
        cand = jnp.where(best_val == m[:, None],
                         best_gid * 128.0 + lane, jnp.float32(1e9))
        gr = (TB // 128) * pl.program_id(0) + (s * TSB) // 128
        idx_ref[pl.ds(gr, 1), pl.ds((s * TSB) % 128, TSB)] = (
            jnp.min(cand, axis=1).astype(jnp.int32)[None, :])


def _nearest_idx(x, E):
    return pl.pallas_call(
        _argmin_body,
        grid=(N_TB,),
        in_specs=[
            pl.BlockSpec((BB, S, D), lambda i: (i, 0, 0)),
            pl.BlockSpec((V, D), lambda i: (0, 0)),
        ],
        out_specs=[
            pl.BlockSpec((NR, 128), lambda i: (0, 0)),
            pl.BlockSpec((V, 128), lambda i: (0, 0)),
        ],
        out_shape=[
            jax.ShapeDtypeStruct((NR, 128), jnp.int32),
            jax.ShapeDtypeStruct((V, 128), jnp.float32),
        ],
        scratch_shapes=[pltpu.VMEM((V // 128, 128), jnp.float32)],
    )(x, E)


def _sc_gather(epad, idx):
    mesh = plsc.VectorSubcoreMesh(core_axis_name="c", subcore_axis_name="s")

    @functools.partial(
        pl.kernel, mesh=mesh,
        out_type=jax.ShapeDtypeStruct((T, 128), jnp.float32),
        scratch_types=[
            pltpu.VMEM((BPW,), jnp.int32),
            pltpu.VMEM((BPW, 128), jnp.float32),
            pltpu.SemaphoreType.DMA,
        ],
    )
    def gather_k(table_hbm, idx_hbm, out_hbm, idx_v, rows_v, sem):
        w = lax.axis_index("s") * NC + lax.axis_index("c")
        pltpu.sync_copy(idx_hbm.at[pl.ds(w * BPW, BPW)], idx_v)
        copies = [
            pltpu.async_copy(table_hbm.at[idx_v.at[pl.ds(j * CH, CH)]],
                             rows_v.at[pl.ds(j * CH, CH)], sem)
            for j in range(BPW // CH)
        ]
        for cp in copies:
            cp.wait()
        pltpu.sync_copy(rows_v, out_hbm.at[pl.ds(w * BPW, BPW)])

    return gather_k(epad, idx)


def kernel(x, E):
    idx, epad = _nearest_idx(x, E)
    values = _sc_gather(epad, idx.reshape(T))
    return values[:, :D].reshape(B, S, D)
